# baseline (speedup 1.0000x reference)
import functools

import jax
import jax.numpy as jnp
from jax import lax
from jax.experimental import pallas as pl
from jax.experimental.pallas import tpu as pltpu
from jax.experimental.pallas import tpu_sc as plsc

NUM_SEG = 10000
SEG_BLOCK = 128
ROW_CHUNK = 512
SC_CHUNK = 80
NUM_WORKERS = 32


def _sc_gather(ctx_flat, cat_table, leaf_rows):
    n3 = ctx_flat.shape[0]
    d = cat_table.shape[1]
    per_w = n3 // NUM_WORKERS
    B3 = SC_CHUNK * 3
    SA, SB = 112, 128
    n_chunks = per_w // B3
    assert n_chunks % 2 == 1 and n_chunks >= 3 and SA + SB == B3
    mesh = plsc.VectorSubcoreMesh(core_axis_name="c", subcore_axis_name="s")
    scr = ([pltpu.VMEM((B3,), jnp.int32)]
           + [pltpu.VMEM((SA,), jnp.int32), pltpu.VMEM((SB,), jnp.int32)] * 2
           + [pltpu.VMEM((SA, d), jnp.float32),
              pltpu.VMEM((SB, d), jnp.float32)] * 2
           + [pltpu.SemaphoreType.DMA] * 4)

    @functools.partial(
        pl.kernel, mesh=mesh,
        out_type=jax.ShapeDtypeStruct((n3, d), jnp.float32),
        scratch_types=scr)
    def gather_kernel(ctx_h, tab_h, out_h, ctxv, i0a, i0b, i1a, i1b,
                      r0a, r0b, r1a, r1b, s0a, s0b, s1a, s1b):
        ibufs = ((i0a, i0b), (i1a, i1b))
        rbufs = ((r0a, r0b), (r1a, r1b))
        sems = ((s0a, s0b), (s1a, s1b))
        wid = lax.axis_index("s") * 2 + lax.axis_index("c")
        base = wid * per_w

        def stage(c, slot):
            off = base + c * B3
            pltpu.sync_copy(ctx_h.at[pl.ds(off, B3)], ctxv)
            for k in range(B3 // 16):
                pos16 = lax.iota(jnp.int32, 16) + (16 * k)
                col16 = pos16 - lax.div(pos16, 3) * 3
                sel16 = col16 * (2 - col16)
                shifted = ctxv[pl.ds(16 * k, 16)] + sel16 * leaf_rows
                if 16 * k < SA:
                    ibufs[slot][0][pl.ds(16 * k, 16)] = shifted
                else:
                    ibufs[slot][1][pl.ds(16 * k - SA, 16)] = shifted
            pltpu.async_copy(tab_h.at[ibufs[slot][0]], rbufs[slot][0],
                             sems[slot][0])
            pltpu.async_copy(tab_h.at[ibufs[slot][1]], rbufs[slot][1],
                             sems[slot][1])

        def drain(slot):
            pltpu.make_async_copy(tab_h.at[ibufs[slot][0]], rbufs[slot][0],
                                  sems[slot][0]).wait()
            pltpu.make_async_copy(tab_h.at[ibufs[slot][1]], rbufs[slot][1],
                                  sems[slot][1]).wait()

        def writeback(c, slot):
            off = base + c * B3
            pltpu.sync_copy(rbufs[slot][0], out_h.at[pl.ds(off, SA)])
            pltpu.sync_copy(rbufs[slot][1], out_h.at[pl.ds(off + SA, SB)])

        stage(0, 0)

        def outer(g, carry):
            c = 2 * g
            stage(c + 1, 1)
            drain(0)
            writeback(c, 0)
            stage(c + 2, 0)
            drain(1)
            writeback(c + 1, 1)
            return carry

        lax.fori_loop(0, (n_chunks - 1) // 2, outer, 0)
        drain(0)
        writeback(n_chunks - 1, 0)

    return gather_kernel(ctx_flat, cat_table)


def _segment_fused(emb, idx2, bounds, wfct, a2, w_out, b_out2, num_blocks):
    n, cat = emb.shape
    code = wfct.shape[1]
    out_dim = w_out.shape[0]
    seg_pad = num_blocks * SEG_BLOCK
    C = ROW_CHUNK

    def body(bounds_r, e_r, i_r, wfc_r, a_r, wout_r, bout_r, out_r,
             ebuf, ibuf, sem_e, sem_i):
        b = pl.program_id(0)
        r0 = bounds_r[b]
        r1 = bounds_r[b + 1]
        c_lo = r0 // C
        c_hi = lax.div(r1 + C - 1, C)
        seg0 = b * SEG_BLOCK

        def start(c, slot):
            off = c * C
            pltpu.make_async_copy(
                e_r.at[pl.ds(off, C)], ebuf.at[slot], sem_e.at[slot]).start()
            pltpu.make_async_copy(
                i_r.at[:, pl.ds(off, C)], ibuf.at[slot], sem_i.at[slot]).start()

        def wait(c, slot):
            off = c * C
            pltpu.make_async_copy(
                e_r.at[pl.ds(off, C)], ebuf.at[slot], sem_e.at[slot]).wait()
            pltpu.make_async_copy(
                i_r.at[:, pl.ds(off, C)], ibuf.at[slot], sem_i.at[slot]).wait()

        @pl.when(c_lo < c_hi)
        def _():
            start(c_lo, 0)

        def chunk(c, carry):
            m, dnm, acc = carry
            slot = lax.rem(c - c_lo, 2)

            @pl.when(c + 1 < c_hi)
            def _():
                start(c + 1, 1 - slot)

            wait(c, slot)
            z = jnp.dot(ebuf[slot], wfc_r[...],
                        preferred_element_type=jnp.float32)
            h = jnp.tanh(z)
            sc = lax.dot_general(a_r[...], h, (((1,), (1,)), ((), ())),
                                 preferred_element_type=jnp.float32)
            rel = ibuf[slot] - seg0
            rows = lax.broadcasted_iota(jnp.int32, (SEG_BLOCK, C), 0)
            onehot = rel == rows
            mc = jnp.max(jnp.where(onehot, sc, -1e30), axis=1, keepdims=True)
            m_new = jnp.maximum(m, mc)
            alpha = jnp.exp(m - m_new)
            ex = jnp.exp(jnp.where(onehot, sc - m_new, -1e30))
            dnm = dnm * alpha + jnp.sum(ex, axis=1, keepdims=True)
            acc = acc * alpha + jnp.dot(
                ex, h, preferred_element_type=jnp.float32)
            return m_new, dnm, acc

        m0 = jnp.full((SEG_BLOCK, 1), -1e30, jnp.float32)
        d0 = jnp.zeros((SEG_BLOCK, 1), jnp.float32)
        a0 = jnp.zeros((SEG_BLOCK, code), jnp.float32)
        m, dnm, acc = lax.fori_loop(c_lo, c_hi, chunk, (m0, d0, a0))
        v = jnp.where(dnm > 0, acc / jnp.where(dnm > 0, dnm, 1.0), 0.0)
        out = lax.dot_general(
            v, wout_r[...], (((1,), (1,)), ((), ())),
            preferred_element_type=jnp.float32)
        out_r[...] = out + bout_r[...]

    return pl.pallas_call(
        body,
        grid=(num_blocks,),
        in_specs=[
            pl.BlockSpec(memory_space=pltpu.MemorySpace.SMEM),
            pl.BlockSpec(memory_space=pltpu.MemorySpace.HBM),
            pl.BlockSpec(memory_space=pltpu.MemorySpace.HBM),
            pl.BlockSpec((cat, code), lambda b: (0, 0)),
            pl.BlockSpec((1, code), lambda b: (0, 0)),
            pl.BlockSpec((out_dim, code), lambda b: (0, 0)),
            pl.BlockSpec((1, out_dim), lambda b: (0, 0)),
        ],
        out_specs=pl.BlockSpec((SEG_BLOCK, out_dim), lambda b: (b, 0)),
        out_shape=jax.ShapeDtypeStruct((seg_pad, out_dim), jnp.float32),
        scratch_shapes=[
            pltpu.VMEM((2, C, cat), jnp.float32),
            pltpu.VMEM((2, 1, C), jnp.int32),
            pltpu.SemaphoreType.DMA((2,)),
            pltpu.SemaphoreType.DMA((2,)),
        ],
    )(bounds, emb, idx2, wfct, a2, w_out, b_out2)


def kernel(contexts, indices, leaf_table, path_table, W_fc, a, W_out, b_out):
    n = contexts.shape[0]
    d = leaf_table.shape[1]
    num_blocks = (NUM_SEG + SEG_BLOCK - 1) // SEG_BLOCK

    cat_table = jnp.concatenate([leaf_table, path_table], axis=0)
    emb3 = _sc_gather(contexts.reshape(-1), cat_table, leaf_table.shape[0])
    emb = emb3.reshape(n, 3 * d)

    seg_starts = jnp.arange(num_blocks, dtype=jnp.int32) * SEG_BLOCK
    bounds = jnp.concatenate([
        jnp.searchsorted(indices, seg_starts).astype(jnp.int32),
        jnp.array([n], jnp.int32),
    ])
    out_full = _segment_fused(emb, indices[None, :], bounds,
                              W_fc.T, a[None, :],
                              W_out, b_out[None, :], num_blocks)
    return out_full[:NUM_SEG]


# speedup vs baseline: 1.4133x; 1.4133x over previous
import functools

import jax
import jax.numpy as jnp
from jax import lax
from jax.experimental import pallas as pl
from jax.experimental.pallas import tpu as pltpu
from jax.experimental.pallas import tpu_sc as plsc

NUM_SEG = 10000
SEG_BLOCK = 128
ROW_CHUNK = 512
SC_CHUNK = 80
NUM_WORKERS = 32


def _sc_gather(c0, c1, c2, leaf_table, path_table):
    n = c0.shape[0]
    d = leaf_table.shape[1]
    per_w = n // NUM_WORKERS
    B = SC_CHUNK
    n_chunks = per_w // B
    assert n_chunks % 2 == 1 and n_chunks >= 3
    mesh = plsc.VectorSubcoreMesh(core_axis_name="c", subcore_axis_name="s")
    row_t = jax.ShapeDtypeStruct((n, d), jnp.float32)
    scr = ([pltpu.VMEM((B,), jnp.int32)] * 6
           + [pltpu.VMEM((B, d), jnp.float32)] * 6
           + [pltpu.SemaphoreType.DMA] * 6)

    @functools.partial(pl.kernel, mesh=mesh, out_type=(row_t, row_t, row_t),
                       scratch_types=scr)
    def gather_kernel(c0_h, c1_h, c2_h, leaf_h, path_h, o0_h, o1_h, o2_h,
                      i00, i01, i02, i10, i11, i12,
                      r00, r01, r02, r10, r11, r12,
                      s00, s01, s02, s10, s11, s12):
        ibufs = ((i00, i01, i02), (i10, i11, i12))
        rbufs = ((r00, r01, r02), (r10, r11, r12))
        sems = ((s00, s01, s02), (s10, s11, s12))
        idxs = (c0_h, c1_h, c2_h)
        tabs = (leaf_h, path_h, leaf_h)
        outs = (o0_h, o1_h, o2_h)
        wid = lax.axis_index("s") * 2 + lax.axis_index("c")
        base = wid * per_w

        def stage(c, slot):
            off = base + c * B
            for t in range(3):
                pltpu.sync_copy(idxs[t].at[pl.ds(off, B)], ibufs[slot][t])
            for t in range(3):
                pltpu.async_copy(tabs[t].at[ibufs[slot][t]], rbufs[slot][t],
                                 sems[slot][t])

        def drain(slot):
            for t in range(3):
                pltpu.make_async_copy(tabs[t].at[ibufs[slot][t]],
                                      rbufs[slot][t], sems[slot][t]).wait()

        def writeback(c, slot):
            off = base + c * B
            for t in range(3):
                pltpu.sync_copy(rbufs[slot][t], outs[t].at[pl.ds(off, B)])

        stage(0, 0)

        def outer(g, carry):
            c = 2 * g
            stage(c + 1, 1)
            drain(0)
            writeback(c, 0)
            stage(c + 2, 0)
            drain(1)
            writeback(c + 1, 1)
            return carry

        lax.fori_loop(0, (n_chunks - 1) // 2, outer, 0)
        drain(0)
        writeback(n_chunks - 1, 0)

    return gather_kernel(c0, c1, c2, leaf_table, path_table)


def _segment_fused(ll, pm, lr, idx2, bounds, w1t, w2t, w3t, a2, w_out,
                   b_out2, num_blocks):
    n, d = ll.shape
    code = w1t.shape[1]
    out_dim = w_out.shape[0]
    seg_pad = num_blocks * SEG_BLOCK
    C = ROW_CHUNK

    def body(bounds_r, ll_r, pm_r, lr_r, i_r, w1_r, w2_r, w3_r, a_r,
             wout_r, bout_r, out_r,
             lbuf, pbuf, rbuf, ibuf, sem_l, sem_p, sem_r, sem_i):
        b = pl.program_id(0)
        r0 = bounds_r[b]
        r1 = bounds_r[b + 1]
        c_lo = r0 // C
        c_hi = lax.div(r1 + C - 1, C)
        seg0 = b * SEG_BLOCK

        def start(c, slot):
            off = c * C
            pltpu.make_async_copy(
                ll_r.at[pl.ds(off, C)], lbuf.at[slot], sem_l.at[slot]).start()
            pltpu.make_async_copy(
                pm_r.at[pl.ds(off, C)], pbuf.at[slot], sem_p.at[slot]).start()
            pltpu.make_async_copy(
                lr_r.at[pl.ds(off, C)], rbuf.at[slot], sem_r.at[slot]).start()
            pltpu.make_async_copy(
                i_r.at[:, pl.ds(off, C)], ibuf.at[slot], sem_i.at[slot]).start()

        def wait(c, slot):
            off = c * C
            pltpu.make_async_copy(
                ll_r.at[pl.ds(off, C)], lbuf.at[slot], sem_l.at[slot]).wait()
            pltpu.make_async_copy(
                pm_r.at[pl.ds(off, C)], pbuf.at[slot], sem_p.at[slot]).wait()
            pltpu.make_async_copy(
                lr_r.at[pl.ds(off, C)], rbuf.at[slot], sem_r.at[slot]).wait()
            pltpu.make_async_copy(
                i_r.at[:, pl.ds(off, C)], ibuf.at[slot], sem_i.at[slot]).wait()

        @pl.when(c_lo < c_hi)
        def _():
            start(c_lo, 0)

        def chunk(c, carry):
            m, dnm, acc = carry
            slot = lax.rem(c - c_lo, 2)

            @pl.when(c + 1 < c_hi)
            def _():
                start(c + 1, 1 - slot)

            wait(c, slot)
            z = jnp.dot(lbuf[slot], w1_r[...],
                        preferred_element_type=jnp.float32)
            z = z + jnp.dot(pbuf[slot], w2_r[...],
                            preferred_element_type=jnp.float32)
            z = z + jnp.dot(rbuf[slot], w3_r[...],
                            preferred_element_type=jnp.float32)
            h = jnp.tanh(z)
            sc = lax.dot_general(a_r[...], h, (((1,), (1,)), ((), ())),
                                 preferred_element_type=jnp.float32)
            rel = ibuf[slot] - seg0
            rows = lax.broadcasted_iota(jnp.int32, (SEG_BLOCK, C), 0)
            onehot = rel == rows
            mc = jnp.max(jnp.where(onehot, sc, -1e30), axis=1, keepdims=True)
            m_new = jnp.maximum(m, mc)
            alpha = jnp.exp(m - m_new)
            ex = jnp.exp(jnp.where(onehot, sc - m_new, -1e30))
            dnm = dnm * alpha + jnp.sum(ex, axis=1, keepdims=True)
            acc = acc * alpha + jnp.dot(
                ex, h, preferred_element_type=jnp.float32)
            return m_new, dnm, acc

        m0 = jnp.full((SEG_BLOCK, 1), -1e30, jnp.float32)
        d0 = jnp.zeros((SEG_BLOCK, 1), jnp.float32)
        a0 = jnp.zeros((SEG_BLOCK, code), jnp.float32)
        m, dnm, acc = lax.fori_loop(c_lo, c_hi, chunk, (m0, d0, a0))
        v = jnp.where(dnm > 0, acc / jnp.where(dnm > 0, dnm, 1.0), 0.0)
        out = lax.dot_general(
            v, wout_r[...], (((1,), (1,)), ((), ())),
            preferred_element_type=jnp.float32)
        out_r[...] = out + bout_r[...]

    return pl.pallas_call(
        body,
        grid=(num_blocks,),
        in_specs=[
            pl.BlockSpec(memory_space=pltpu.MemorySpace.SMEM),
            pl.BlockSpec(memory_space=pltpu.MemorySpace.HBM),
            pl.BlockSpec(memory_space=pltpu.MemorySpace.HBM),
            pl.BlockSpec(memory_space=pltpu.MemorySpace.HBM),
            pl.BlockSpec(memory_space=pltpu.MemorySpace.HBM),
            pl.BlockSpec((d, code), lambda b: (0, 0)),
            pl.BlockSpec((d, code), lambda b: (0, 0)),
            pl.BlockSpec((d, code), lambda b: (0, 0)),
            pl.BlockSpec((1, code), lambda b: (0, 0)),
            pl.BlockSpec((out_dim, code), lambda b: (0, 0)),
            pl.BlockSpec((1, out_dim), lambda b: (0, 0)),
        ],
        out_specs=pl.BlockSpec((SEG_BLOCK, out_dim), lambda b: (b, 0)),
        out_shape=jax.ShapeDtypeStruct((seg_pad, out_dim), jnp.float32),
        scratch_shapes=[
            pltpu.VMEM((2, C, d), jnp.float32),
            pltpu.VMEM((2, C, d), jnp.float32),
            pltpu.VMEM((2, C, d), jnp.float32),
            pltpu.VMEM((2, 1, C), jnp.int32),
            pltpu.SemaphoreType.DMA((2,)),
            pltpu.SemaphoreType.DMA((2,)),
            pltpu.SemaphoreType.DMA((2,)),
            pltpu.SemaphoreType.DMA((2,)),
        ],
    )(bounds, ll, pm, lr, idx2, w1t, w2t, w3t, a2, w_out, b_out2)


def kernel(contexts, indices, leaf_table, path_table, W_fc, a, W_out, b_out):
    n = contexts.shape[0]
    d = leaf_table.shape[1]
    num_blocks = (NUM_SEG + SEG_BLOCK - 1) // SEG_BLOCK

    ll, pm, lr = _sc_gather(contexts[:, 0], contexts[:, 1], contexts[:, 2],
                            leaf_table, path_table)

    wt = W_fc.T
    seg_starts = jnp.arange(num_blocks, dtype=jnp.int32) * SEG_BLOCK
    bounds = jnp.concatenate([
        jnp.searchsorted(indices, seg_starts).astype(jnp.int32),
        jnp.array([n], jnp.int32),
    ])
    out_full = _segment_fused(ll, pm, lr, indices[None, :], bounds,
                              wt[:d], wt[d:2 * d], wt[2 * d:], a[None, :],
                              W_out, b_out[None, :], num_blocks)
    return out_full[:NUM_SEG]


# speedup vs baseline: 1.5900x; 1.1251x over previous
import functools

import jax
import jax.numpy as jnp
from jax import lax
from jax.experimental import pallas as pl
from jax.experimental.pallas import tpu as pltpu
from jax.experimental.pallas import tpu_sc as plsc

NUM_SEG = 10000
SEG_BLOCK = 128
ROW_CHUNK = 1024
SC_CHUNK = 80
NUM_WORKERS = 32


def _sc_gather(c0, c1, c2, leaf_table, path_table):
    n = c0.shape[0]
    d = leaf_table.shape[1]
    per_w = n // NUM_WORKERS
    B = SC_CHUNK
    n_chunks = per_w // B
    assert n_chunks % 2 == 1 and n_chunks >= 3
    mesh = plsc.VectorSubcoreMesh(core_axis_name="c", subcore_axis_name="s")
    row_t = jax.ShapeDtypeStruct((n, d), jnp.float32)
    scr = ([pltpu.VMEM((B,), jnp.int32)] * 6
           + [pltpu.VMEM((B, d), jnp.float32)] * 6
           + [pltpu.SemaphoreType.DMA] * 6)

    @functools.partial(pl.kernel, mesh=mesh, out_type=(row_t, row_t, row_t),
                       scratch_types=scr)
    def gather_kernel(c0_h, c1_h, c2_h, leaf_h, path_h, o0_h, o1_h, o2_h,
                      i00, i01, i02, i10, i11, i12,
                      r00, r01, r02, r10, r11, r12,
                      s00, s01, s02, s10, s11, s12):
        ibufs = ((i00, i01, i02), (i10, i11, i12))
        rbufs = ((r00, r01, r02), (r10, r11, r12))
        sems = ((s00, s01, s02), (s10, s11, s12))
        idxs = (c0_h, c1_h, c2_h)
        tabs = (leaf_h, path_h, leaf_h)
        outs = (o0_h, o1_h, o2_h)
        wid = lax.axis_index("s") * 2 + lax.axis_index("c")
        base = wid * per_w

        def stage(c, slot):
            off = base + c * B
            for t in range(3):
                pltpu.sync_copy(idxs[t].at[pl.ds(off, B)], ibufs[slot][t])
            for t in range(3):
                pltpu.async_copy(tabs[t].at[ibufs[slot][t]], rbufs[slot][t],
                                 sems[slot][t])

        def drain(slot):
            for t in range(3):
                pltpu.make_async_copy(tabs[t].at[ibufs[slot][t]],
                                      rbufs[slot][t], sems[slot][t]).wait()

        def writeback(c, slot):
            off = base + c * B
            for t in range(3):
                pltpu.sync_copy(rbufs[slot][t], outs[t].at[pl.ds(off, B)])

        stage(0, 0)

        def outer(g, carry):
            c = 2 * g
            stage(c + 1, 1)
            drain(0)
            writeback(c, 0)
            stage(c + 2, 0)
            drain(1)
            writeback(c + 1, 1)
            return carry

        lax.fori_loop(0, (n_chunks - 1) // 2, outer, 0)
        drain(0)
        writeback(n_chunks - 1, 0)

    return gather_kernel(c0, c1, c2, leaf_table, path_table)


def _segment_fused(ll, pm, lr, idx2, bounds, w1t, w2t, w3t, a2, w_out,
                   b_out2, num_blocks):
    n, d = ll.shape
    code = w1t.shape[1]
    out_dim = w_out.shape[0]
    seg_pad = num_blocks * SEG_BLOCK
    C = ROW_CHUNK

    def body(bounds_r, ll_r, pm_r, lr_r, i_r, w1_r, w2_r, w3_r, a_r,
             wout_r, bout_r, out_r,
             lbuf, pbuf, rbuf, ibuf, sem_l, sem_p, sem_r, sem_i):
        b = pl.program_id(0)
        r0 = bounds_r[b]
        r1 = bounds_r[b + 1]
        c_lo = r0 // C
        c_hi = lax.div(r1 + C - 1, C)
        seg0 = b * SEG_BLOCK

        def start(c, slot):
            off = c * C
            pltpu.make_async_copy(
                ll_r.at[pl.ds(off, C)], lbuf.at[slot], sem_l.at[slot]).start()
            pltpu.make_async_copy(
                pm_r.at[pl.ds(off, C)], pbuf.at[slot], sem_p.at[slot]).start()
            pltpu.make_async_copy(
                lr_r.at[pl.ds(off, C)], rbuf.at[slot], sem_r.at[slot]).start()
            pltpu.make_async_copy(
                i_r.at[:, pl.ds(off, C)], ibuf.at[slot], sem_i.at[slot]).start()

        def wait(c, slot):
            off = c * C
            pltpu.make_async_copy(
                ll_r.at[pl.ds(off, C)], lbuf.at[slot], sem_l.at[slot]).wait()
            pltpu.make_async_copy(
                pm_r.at[pl.ds(off, C)], pbuf.at[slot], sem_p.at[slot]).wait()
            pltpu.make_async_copy(
                lr_r.at[pl.ds(off, C)], rbuf.at[slot], sem_r.at[slot]).wait()
            pltpu.make_async_copy(
                i_r.at[:, pl.ds(off, C)], ibuf.at[slot], sem_i.at[slot]).wait()

        @pl.when(c_lo < c_hi)
        def _():
            start(c_lo, 0)

        def chunk(c, carry):
            m, dnm, acc = carry
            slot = lax.rem(c - c_lo, 2)

            @pl.when(c + 1 < c_hi)
            def _():
                start(c + 1, 1 - slot)

            wait(c, slot)
            z = jnp.dot(lbuf[slot], w1_r[...],
                        preferred_element_type=jnp.float32)
            z = z + jnp.dot(pbuf[slot], w2_r[...],
                            preferred_element_type=jnp.float32)
            z = z + jnp.dot(rbuf[slot], w3_r[...],
                            preferred_element_type=jnp.float32)
            h = jnp.tanh(z)
            sc = lax.dot_general(a_r[...], h, (((1,), (1,)), ((), ())),
                                 preferred_element_type=jnp.float32)
            rel = ibuf[slot] - seg0
            rows = lax.broadcasted_iota(jnp.int32, (SEG_BLOCK, C), 0)
            onehot = rel == rows
            mc = jnp.max(jnp.where(onehot, sc, -1e30), axis=1, keepdims=True)
            m_new = jnp.maximum(m, mc)
            alpha = jnp.exp(m - m_new)
            ex = jnp.exp(jnp.where(onehot, sc - m_new, -1e30))
            dnm = dnm * alpha + jnp.sum(ex, axis=1, keepdims=True)
            acc = acc * alpha + jnp.dot(
                ex, h, preferred_element_type=jnp.float32)
            return m_new, dnm, acc

        m0 = jnp.full((SEG_BLOCK, 1), -1e30, jnp.float32)
        d0 = jnp.zeros((SEG_BLOCK, 1), jnp.float32)
        a0 = jnp.zeros((SEG_BLOCK, code), jnp.float32)
        m, dnm, acc = lax.fori_loop(c_lo, c_hi, chunk, (m0, d0, a0))
        v = jnp.where(dnm > 0, acc / jnp.where(dnm > 0, dnm, 1.0), 0.0)
        out = lax.dot_general(
            v, wout_r[...], (((1,), (1,)), ((), ())),
            preferred_element_type=jnp.float32)
        out_r[...] = out + bout_r[...]

    return pl.pallas_call(
        body,
        grid=(num_blocks,),
        in_specs=[
            pl.BlockSpec(memory_space=pltpu.MemorySpace.SMEM),
            pl.BlockSpec(memory_space=pltpu.MemorySpace.HBM),
            pl.BlockSpec(memory_space=pltpu.MemorySpace.HBM),
            pl.BlockSpec(memory_space=pltpu.MemorySpace.HBM),
            pl.BlockSpec(memory_space=pltpu.MemorySpace.HBM),
            pl.BlockSpec((d, code), lambda b: (0, 0)),
            pl.BlockSpec((d, code), lambda b: (0, 0)),
            pl.BlockSpec((d, code), lambda b: (0, 0)),
            pl.BlockSpec((1, code), lambda b: (0, 0)),
            pl.BlockSpec((out_dim, code), lambda b: (0, 0)),
            pl.BlockSpec((1, out_dim), lambda b: (0, 0)),
        ],
        out_specs=pl.BlockSpec((SEG_BLOCK, out_dim), lambda b: (b, 0)),
        out_shape=jax.ShapeDtypeStruct((seg_pad, out_dim), jnp.float32),
        scratch_shapes=[
            pltpu.VMEM((2, C, d), jnp.float32),
            pltpu.VMEM((2, C, d), jnp.float32),
            pltpu.VMEM((2, C, d), jnp.float32),
            pltpu.VMEM((2, 1, C), jnp.int32),
            pltpu.SemaphoreType.DMA((2,)),
            pltpu.SemaphoreType.DMA((2,)),
            pltpu.SemaphoreType.DMA((2,)),
            pltpu.SemaphoreType.DMA((2,)),
        ],
    )(bounds, ll, pm, lr, idx2, w1t, w2t, w3t, a2, w_out, b_out2)


def kernel(contexts, indices, leaf_table, path_table, W_fc, a, W_out, b_out):
    n = contexts.shape[0]
    d = leaf_table.shape[1]
    num_blocks = (NUM_SEG + SEG_BLOCK - 1) // SEG_BLOCK

    ll, pm, lr = _sc_gather(contexts[:, 0], contexts[:, 1], contexts[:, 2],
                            leaf_table, path_table)

    wt = W_fc.T
    seg_starts = jnp.arange(num_blocks, dtype=jnp.int32) * SEG_BLOCK
    bounds = jnp.concatenate([
        jnp.searchsorted(indices, seg_starts).astype(jnp.int32),
        jnp.array([n], jnp.int32),
    ])
    out_full = _segment_fused(ll, pm, lr, indices[None, :], bounds,
                              wt[:d], wt[d:2 * d], wt[2 * d:], a[None, :],
                              W_out, b_out[None, :], num_blocks)
    return out_full[:NUM_SEG]
